# baseline (speedup 1.0000x reference)
import functools
import math

import jax
import jax.numpy as jnp
from jax import lax
from jax.experimental import pallas as pl
from jax.experimental.pallas import tpu as pltpu
from jax.experimental.pallas import tpu_sc as plsc

D_MODEL = 128
GRP = 128
LANES = 16


@functools.cache
def _make_gather(n_groups_total: int, vocab: int, d: int, nc: int, ns: int):
    nw = nc * ns
    g_per_w = n_groups_total // nw
    mesh = plsc.VectorSubcoreMesh(core_axis_name="c", subcore_axis_name="s")

    NBUF = 5
    LOOKAHEAD = 3
    assert g_per_w % NBUF == 0

    @functools.partial(
        pl.kernel,
        out_type=jax.ShapeDtypeStruct((n_groups_total * GRP, d), jnp.float32),
        mesh=mesh,
        scratch_types=[
            pltpu.VMEM((g_per_w, GRP), jnp.int32),
            pltpu.VMEM((NBUF, GRP, d), jnp.float32),
            [pltpu.SemaphoreType.DMA] * NBUF,
            [pltpu.SemaphoreType.DMA] * NBUF,
        ],
    )
    def gather_kernel(idx_hbm, lut_hbm, out_hbm, idx_v, rows_v, gsem, wsem):
        cid = lax.axis_index("c")
        sid = lax.axis_index("s")
        wid = sid * nc + cid
        g_base = wid * g_per_w

        pltpu.sync_copy(idx_hbm.at[wid], idx_v)

        scale = jnp.float32(math.sqrt(d))
        vecs_per_row = d // LANES

        def fire_gather(j, slot):
            pltpu.async_copy(
                lut_hbm.at[idx_v.at[j]], rows_v.at[slot], gsem[slot])

        def out_copy(j, slot, sem):
            return pltpu.make_async_copy(
                rows_v.at[slot], out_hbm.at[pl.ds((g_base + j) * GRP, GRP)],
                sem)

        for j in range(LOOKAHEAD):
            fire_gather(j, j % NBUF)

        @pl.loop(0, g_per_w, step=NBUF)
        def _ring(j0):
            for b in range(NBUF):
                j = j0 + b

                pltpu.make_async_copy(
                    lut_hbm.at[idx_v.at[j]], rows_v.at[b], gsem[b]).wait()

                @pl.loop(0, GRP)
                def _row(r):
                    for k in range(vecs_per_row):
                        sl = pl.ds(k * LANES, LANES)
                        rows_v[b, r, sl] = rows_v[b, r, sl] * scale

                out_copy(j, b, wsem[b]).start()

                nslot = (b + LOOKAHEAD) % NBUF
                jn = j + LOOKAHEAD
                jold = j - (NBUF - LOOKAHEAD)

                @pl.when(jnp.logical_and(jold >= 0, jn < g_per_w))
                def _drain_old_write():
                    out_copy(jold, nslot, wsem[nslot]).wait()

                @pl.when(jn < g_per_w)
                def _refill():
                    fire_gather(jn, nslot)

        for k in range(NBUF):
            j = g_per_w - NBUF + k
            out_copy(j, j % NBUF, wsem[j % NBUF]).wait()

    return gather_kernel


def kernel(x, lut):
    vocab, d = lut.shape
    n = x.size
    info = plsc.get_sparse_core_info()
    nc, ns = info.num_cores, info.num_subcores
    nw = nc * ns
    assert n % (nw * GRP) == 0
    n_groups = n // GRP
    idx3d = x.reshape(nw, n_groups // nw, GRP).astype(jnp.int32)
    out = _make_gather(n_groups, vocab, d, nc, ns)(idx3d, lut)
    return out.reshape(x.shape + (d,))


# speedup vs baseline: 1.0113x; 1.0113x over previous
import functools
import math

import jax
import jax.numpy as jnp
from jax import lax
from jax.experimental import pallas as pl
from jax.experimental.pallas import tpu as pltpu
from jax.experimental.pallas import tpu_sc as plsc

D_MODEL = 128
GRP = 128
LANES = 16


@functools.cache
def _make_gather(n_groups_total: int, vocab: int, d: int, nc: int, ns: int):
    nw = nc * ns
    g_per_w = n_groups_total // nw
    mesh = plsc.VectorSubcoreMesh(core_axis_name="c", subcore_axis_name="s")

    NBUF = 7
    LOOKAHEAD = 5
    padded = ((g_per_w + NBUF - 1) // NBUF) * NBUF

    @functools.partial(
        pl.kernel,
        out_type=jax.ShapeDtypeStruct((n_groups_total * GRP, d), jnp.float32),
        mesh=mesh,
        scratch_types=[
            pltpu.VMEM((g_per_w, GRP), jnp.int32),
            pltpu.VMEM((NBUF, GRP, d), jnp.float32),
            [pltpu.SemaphoreType.DMA] * NBUF,
            [pltpu.SemaphoreType.DMA] * NBUF,
        ],
    )
    def gather_kernel(idx_hbm, lut_hbm, out_hbm, idx_v, rows_v, gsem, wsem):
        cid = lax.axis_index("c")
        sid = lax.axis_index("s")
        wid = sid * nc + cid
        g_base = wid * g_per_w

        pltpu.sync_copy(idx_hbm.at[wid], idx_v)

        scale = jnp.float32(math.sqrt(d))
        vecs_per_row = d // LANES

        def fire_gather(j, slot):
            pltpu.async_copy(
                lut_hbm.at[idx_v.at[j]], rows_v.at[slot], gsem[slot])

        def out_copy(j, slot, sem):
            return pltpu.make_async_copy(
                rows_v.at[slot], out_hbm.at[pl.ds((g_base + j) * GRP, GRP)],
                sem)

        for j in range(LOOKAHEAD):
            fire_gather(j, j % NBUF)

        @pl.loop(0, padded, step=NBUF)
        def _ring(j0):
            for b in range(NBUF):
                j = j0 + b

                @pl.when(j < g_per_w)
                def _process():
                    pltpu.make_async_copy(
                        lut_hbm.at[idx_v.at[j]], rows_v.at[b], gsem[b]).wait()

                    @pl.loop(0, GRP)
                    def _row(r):
                        for k in range(vecs_per_row):
                            sl = pl.ds(k * LANES, LANES)
                            rows_v[b, r, sl] = rows_v[b, r, sl] * scale

                    out_copy(j, b, wsem[b]).start()

                    nslot = (b + LOOKAHEAD) % NBUF
                    jn = j + LOOKAHEAD
                    jold = j - (NBUF - LOOKAHEAD)

                    @pl.when(jnp.logical_and(jold >= 0, jn < g_per_w))
                    def _drain_old_write():
                        out_copy(jold, nslot, wsem[nslot]).wait()

                    @pl.when(jn < g_per_w)
                    def _refill():
                        fire_gather(jn, nslot)

        for k in range(NBUF):
            j = g_per_w - NBUF + k
            out_copy(j, j % NBUF, wsem[j % NBUF]).wait()

    return gather_kernel


def kernel(x, lut):
    vocab, d = lut.shape
    n = x.size
    info = plsc.get_sparse_core_info()
    nc, ns = info.num_cores, info.num_subcores
    nw = nc * ns
    assert n % (nw * GRP) == 0
    n_groups = n // GRP
    idx3d = x.reshape(nw, n_groups // nw, GRP).astype(jnp.int32)
    out = _make_gather(n_groups, vocab, d, nc, ns)(idx3d, lut)
    return out.reshape(x.shape + (d,))
